# Initial kernel scaffold; baseline (speedup 1.0000x reference)
import jax
import jax.numpy as jnp
from jax.experimental import pallas as pl


def kernel(epoch, indices, logits, p):
    raise NotImplementedError("write your pallas kernel here")


# speedup vs baseline: 1.1302x; 1.1302x over previous
import functools

import jax
import jax.numpy as jnp
from jax import lax
from jax.experimental import pallas as pl
from jax.experimental.pallas import tpu as pltpu
from jax.experimental.pallas import tpu_sc as plsc

_B = 16384
_D = 128
_L = 16
_NC = 2
_NS = 16
_NW = _NC * _NS
_BPW = _B // _NW
_CHUNK = 128
_NCHUNK = _BPW // _CHUNK
_DV = _D // _L


def _bcast_last_lane(v, lane15):
    dn = lax.GatherDimensionNumbers(
        offset_dims=(), collapsed_slice_dims=(0,), start_index_map=(0,)
    )
    return lax.gather(
        v, lane15[:, None], dn, (1,),
        mode=lax.GatherScatterMode.PROMISE_IN_BOUNDS,
    )


def _sc_body(idx_hbm, logits_hbm, table_hbm, out_hbm, idx_v, rows_v, log_v, accw_v, sem):
    wid = lax.axis_index("s") * _NC + lax.axis_index("c")
    base = wid * _BPW

    def chunk_body(ch, acc):
        off = base + ch * _CHUNK
        pltpu.sync_copy(idx_hbm.at[pl.ds(off, _CHUNK)], idx_v)
        gat = pltpu.async_copy(table_hbm.at[idx_v], rows_v, sem)
        pltpu.sync_copy(logits_hbm.at[pl.ds(off, _CHUNK)], log_v)
        gat.wait()

        lane15 = jnp.full((_L,), _L - 1, jnp.int32)

        def row_body(r, racc):
            lv = [log_v[r, pl.ds(16 * j, 16)] for j in range(_DV)]
            m = lv[0]
            for j in range(1, _DV):
                m = jnp.maximum(m, lv[j])
            ms = _bcast_last_lane(plsc.cummax(m), lane15)
            ev = [jnp.exp(lv[j] - ms) for j in range(_DV)]
            s = ev[0]
            for j in range(1, _DV):
                s = s + ev[j]
            sv = _bcast_last_lane(plsc.cumsum(s), lane15)
            inv = 1.0 / sv
            for j in range(_DV):
                d = rows_v[r, pl.ds(16 * j, 16)] - ev[j] * inv
                racc = racc + d * d
            return racc

        return lax.fori_loop(0, _CHUNK, row_body, acc)

    acc = lax.fori_loop(0, _NCHUNK, chunk_body, jnp.zeros((_L,), jnp.float32))
    accw_v[...] = acc
    pltpu.sync_copy(accw_v, out_hbm.at[pl.ds(wid * _L, _L)])


@jax.jit
def _sc_partials(indices, logits, p):
    mesh = plsc.VectorSubcoreMesh(
        core_axis_name="c", subcore_axis_name="s", num_cores=_NC, num_subcores=_NS
    )
    return pl.kernel(
        _sc_body,
        out_type=jax.ShapeDtypeStruct((_NW * _L,), jnp.float32),
        mesh=mesh,
        scratch_types=[
            pltpu.VMEM((_CHUNK,), jnp.int32),
            pltpu.VMEM((_CHUNK, _D), jnp.float32),
            pltpu.VMEM((_CHUNK, _D), jnp.float32),
            pltpu.VMEM((_L,), jnp.float32),
            pltpu.SemaphoreType.DMA,
        ],
        compiler_params=pltpu.CompilerParams(needs_layout_passes=False),
    )(indices, logits, p)


def kernel(epoch, indices, logits, p):
    partials = _sc_partials(indices, logits, p)
    phase = 1.0 - (epoch - 0.0) / 50.0
    ramp = jnp.exp(-5.0 * phase * phase)
    w = jnp.where(epoch < 0, 0.0, jnp.where(epoch > 50, 1.0, ramp))
    return jnp.sum(partials) * w / _B
